# baseline (speedup 1.0000x reference)
import jax
import jax.numpy as jnp
from jax.experimental import pallas as pl

_NUM_SYM = 14
_FFT = 4096
_N = 2
_BATCH = 128
_TIN = 384
_TOUT = 448
_TS = 32

_BB = 32

_DMAP = (0, 1, 1, 2, 3, 4, 5, 6, 7, 8, 9, 9, 10, 11)


def _body(x_ref, p_ref, o_ref):
    s = pl.program_id(1)
    is_pilot = jnp.logical_or(s == 2, s == 11)

    @pl.when(jnp.logical_not(is_pilot))
    def _copy():
        o_ref[...] = x_ref[...]

    @pl.when(is_pilot)
    def _pilot():
        base = jnp.where(s == 2, 0, _TS)
        p = p_ref[pl.ds(base, _TS)]
        row = jnp.concatenate([p, p], axis=1)
        o_ref[...] = jnp.broadcast_to(row[None], (_BB, _TS, 256))


def kernel(inputs, pilots):
    b = inputs.shape[0]
    x = inputs.reshape(b, _TIN, 128, _N).transpose(0, 1, 3, 2).reshape(b, _TIN, _N * 128)
    p = pilots.reshape(64, 128)
    out = pl.pallas_call(
        _body,
        grid=(b // _BB, _NUM_SYM),
        in_specs=[
            pl.BlockSpec((_BB, _TS, _N * 128),
                         lambda i, s: (i, s - (s >= 2) - (s >= 11), 0)),
            pl.BlockSpec((64, 128), lambda i, s: (0, 0)),
        ],
        out_specs=pl.BlockSpec((_BB, _TS, _N * 128), lambda i, s: (i, s, 0)),
        out_shape=jax.ShapeDtypeStruct((b, _TOUT, _N * 128), inputs.dtype),
    )(x, p)
    return (out.reshape(b, _TOUT, _N, 128)
               .transpose(0, 1, 3, 2)
               .reshape(b, 1, 1, _NUM_SYM, _FFT, _N))
